# baseline (speedup 1.0000x reference)
import functools

import jax
import jax.numpy as jnp
from jax import lax
from jax.experimental import pallas as pl
from jax.experimental.pallas import tpu as pltpu
from jax.experimental.pallas import tpu_sc as plsc

N_NODES = 10000
N_EDGES = 320000
IN_DIM = 128
HID = 16
OUT_DIM = 40

NC, NS = 2, 16
NW = NC * NS
NPAD = 10112
ROWS_PT = NPAD // NS
EB = 1600
TOTB = N_EDGES // EB
NB_LO = TOTB // NW
NB_HI = NB_LO + 1
W_HI = NW - (TOTB - NW * NB_LO)

_MESH = dict(core_axis_name="c", subcore_axis_name="s",
             num_cores=NC, num_subcores=NS)
_SC_PARAMS = pltpu.CompilerParams(use_tc_tiling_on_sc=False)


DW = 8


@functools.partial(
    pl.kernel,
    out_type=jax.ShapeDtypeStruct((NC, NPAD, DW), jnp.float32),
    mesh=plsc.VectorSubcoreMesh(**_MESH),
    compiler_params=_SC_PARAMS,
    scratch_types=[
        pltpu.VMEM((NB_HI, EB), jnp.int32),
        pltpu.VMEM((EB, DW), jnp.float32),
        pltpu.VMEM_SHARED((NPAD, DW), jnp.float32),
    ],
)
def _deg_kernel(dstf, ones_hbm, zeros_hbm, out, dst_v, ones_v, acc_sh):
    c = lax.axis_index("c")
    s = lax.axis_index("s")
    w = s * NC + c
    base = s * ROWS_PT
    pltpu.sync_copy(zeros_hbm.at[pl.ds(base, ROWS_PT)],
                    acc_sh.at[pl.ds(base, ROWS_PT)])
    off = NB_LO * w + jnp.maximum(0, w - W_HI)
    pltpu.sync_copy(dstf.at[pl.ds(off, NB_HI)], dst_v)
    pltpu.sync_copy(ones_hbm, ones_v)
    plsc.subcore_barrier()

    def body(j, carry):
        pltpu.sync_copy(ones_v, acc_sh.at[dst_v.at[j]], add=True)
        return carry

    nb = NB_LO + jnp.where(w >= W_HI, 1, 0)
    lax.fori_loop(0, nb, body, 0)
    plsc.subcore_barrier()
    pltpu.sync_copy(acc_sh.at[pl.ds(base, ROWS_PT)],
                    out.at[c, pl.ds(base, ROWS_PT)])


def _make_agg_kernel(d):
    @functools.partial(
        pl.kernel,
        out_type=jax.ShapeDtypeStruct((NC, NPAD, d), jnp.float32),
        mesh=plsc.VectorSubcoreMesh(**_MESH),
        compiler_params=_SC_PARAMS,
        scratch_types=[
            pltpu.VMEM((NB_HI, EB), jnp.int32),
            pltpu.VMEM((NB_HI, EB), jnp.int32),
            pltpu.VMEM((EB, d), jnp.float32),
            pltpu.VMEM_SHARED((NPAD, d), jnp.float32),
            pltpu.SemaphoreType.DMA,
        ],
    )
    def _agg(hs_hbm, srcf, dstf, zeros_hbm, out,
             src_v, dst_v, rows_v, acc_sh, sem):
        c = lax.axis_index("c")
        s = lax.axis_index("s")
        w = s * NC + c
        base = s * ROWS_PT
        pltpu.sync_copy(zeros_hbm.at[pl.ds(base, ROWS_PT)],
                        acc_sh.at[pl.ds(base, ROWS_PT)])
        off = NB_LO * w + jnp.maximum(0, w - W_HI)
        pltpu.sync_copy(srcf.at[pl.ds(off, NB_HI)], src_v)
        pltpu.sync_copy(dstf.at[pl.ds(off, NB_HI)], dst_v)
        plsc.subcore_barrier()

        def body(j, carry):
            pltpu.async_copy(hs_hbm.at[src_v.at[j]], rows_v, sem).wait()
            pltpu.sync_copy(rows_v, acc_sh.at[dst_v.at[j]], add=True)
            return carry

        nb = NB_LO + jnp.where(w >= W_HI, 1, 0)
        lax.fori_loop(0, nb, body, 0)
        plsc.subcore_barrier()
        pltpu.sync_copy(acc_sh.at[pl.ds(base, ROWS_PT)],
                        out.at[c, pl.ds(base, ROWS_PT)])

    return _agg


D2 = 48
_agg16 = _make_agg_kernel(HID)
_agg48 = _make_agg_kernel(D2)


_RB = 2000


def _l1_body(x_ref, w_ref, deg_ref, hs_ref, dinv_ref):
    h1 = jnp.dot(x_ref[...], w_ref[...], preferred_element_type=jnp.float32)
    deg = deg_ref[0][:, :1] + deg_ref[1][:, :1] + 1.0
    dinv = lax.rsqrt(deg)
    dinv_ref[...] = dinv
    hs_ref[...] = h1 * dinv


def _mid_body(agg_ref, hs1_ref, dinv_ref, b1_ref, w2_ref, hs2_ref):
    dinv = dinv_ref[...]
    t = (agg_ref[0] + agg_ref[1] + hs1_ref[...]) * dinv + b1_ref[...]
    t = jnp.maximum(t, 0.0)
    hs2_ref[...] = jnp.dot(t, w2_ref[...],
                           preferred_element_type=jnp.float32) * dinv


def _out_body(agg_ref, hs2_ref, dinv_ref, b2_ref, o_ref):
    o_ref[...] = ((agg_ref[0][:, :OUT_DIM] + agg_ref[1][:, :OUT_DIM]
                   + hs2_ref[...][:, :OUT_DIM]) * dinv_ref[...]
                  + b2_ref[...])


def kernel(x, edge_index, W1, b1, W2, b2):
    ei = edge_index.astype(jnp.int32)
    srcf = ei[0].reshape(TOTB, EB)
    dstf = ei[1].reshape(TOTB, EB)

    W2p = jnp.pad(W2, ((0, 0), (0, D2 - OUT_DIM)))
    b1r = b1.reshape(1, HID)
    b2r = b2.reshape(1, OUT_DIM)

    ones_eb = jnp.ones((EB, DW), jnp.float32)
    zeros1 = jnp.zeros((NPAD, DW), jnp.float32)
    zeros16 = jnp.zeros((NPAD, HID), jnp.float32)
    zeros48 = jnp.zeros((NPAD, D2), jnp.float32)

    deg_parts = _deg_kernel(dstf, ones_eb, zeros1)

    hs1, dinv = pl.pallas_call(
        _l1_body,
        grid=(N_NODES // _RB,),
        in_specs=[pl.BlockSpec((_RB, IN_DIM), lambda i: (i, 0)),
                  pl.BlockSpec((IN_DIM, HID), lambda i: (0, 0)),
                  pl.BlockSpec((NC, _RB, DW), lambda i: (0, i, 0))],
        out_specs=[pl.BlockSpec((_RB, HID), lambda i: (i, 0)),
                   pl.BlockSpec((_RB, 1), lambda i: (i, 0))],
        out_shape=[jax.ShapeDtypeStruct((N_NODES, HID), jnp.float32),
                   jax.ShapeDtypeStruct((N_NODES, 1), jnp.float32)],
    )(x, W1, deg_parts)

    agg1 = _agg16(hs1, srcf, dstf, zeros16)

    hs2 = pl.pallas_call(
        _mid_body,
        grid=(N_NODES // _RB,),
        in_specs=[pl.BlockSpec((NC, _RB, HID), lambda i: (0, i, 0)),
                  pl.BlockSpec((_RB, HID), lambda i: (i, 0)),
                  pl.BlockSpec((_RB, 1), lambda i: (i, 0)),
                  pl.BlockSpec((1, HID), lambda i: (0, 0)),
                  pl.BlockSpec((HID, D2), lambda i: (0, 0))],
        out_specs=pl.BlockSpec((_RB, D2), lambda i: (i, 0)),
        out_shape=jax.ShapeDtypeStruct((N_NODES, D2), jnp.float32),
    )(agg1, hs1, dinv, b1r, W2p)

    agg2 = _agg48(hs2, srcf, dstf, zeros48)

    outp = pl.pallas_call(
        _out_body,
        grid=(N_NODES // _RB,),
        in_specs=[pl.BlockSpec((NC, _RB, D2), lambda i: (0, i, 0)),
                  pl.BlockSpec((_RB, D2), lambda i: (i, 0)),
                  pl.BlockSpec((_RB, 1), lambda i: (i, 0)),
                  pl.BlockSpec((1, OUT_DIM), lambda i: (0, 0))],
        out_specs=pl.BlockSpec((_RB, OUT_DIM), lambda i: (i, 0)),
        out_shape=jax.ShapeDtypeStruct((N_NODES, OUT_DIM), jnp.float32),
    )(agg2, hs2, dinv, b2r)

    return outp


# speedup vs baseline: 1.0825x; 1.0825x over previous
import functools

import jax
import jax.numpy as jnp
from jax import lax
from jax.experimental import pallas as pl
from jax.experimental.pallas import tpu as pltpu
from jax.experimental.pallas import tpu_sc as plsc

N_NODES = 10000
N_EDGES = 320000
IN_DIM = 128
HID = 16
OUT_DIM = 40

NC, NS = 2, 16
NW = NC * NS
NPAD = 10112
ROWS_PT = NPAD // NS
EB = 1280
TPB = EB // 128
TOTB = N_EDGES // EB
NB_LO = TOTB // NW
NB_HI = NB_LO + 1
W_HI = NW - (TOTB - NW * NB_LO)

_MESH = dict(core_axis_name="c", subcore_axis_name="s",
             num_cores=NC, num_subcores=NS)
_SC_PARAMS = pltpu.CompilerParams(use_tc_tiling_on_sc=False)


DW = 8


@functools.partial(
    pl.kernel,
    out_type=[jax.ShapeDtypeStruct((NC, NPAD, DW), jnp.float32),
              jax.ShapeDtypeStruct((TOTB, EB), jnp.int32),
              jax.ShapeDtypeStruct((TOTB, EB), jnp.int32)],
    mesh=plsc.VectorSubcoreMesh(**_MESH),
    compiler_params=_SC_PARAMS,
    scratch_types=[
        pltpu.VMEM((NB_HI, EB), jnp.int32),
        pltpu.VMEM((NB_HI, EB), jnp.int32),
        pltpu.VMEM((EB, DW), jnp.float32),
        pltpu.VMEM_SHARED((NPAD, DW), jnp.float32),
        pltpu.SemaphoreType.DMA,
    ],
)
def _deg_kernel(ei3, ones_hbm, zeros_hbm, deg_out, src_lin, dst_lin,
                src_v, dst_v, ones_v, acc_sh, sem):
    c = lax.axis_index("c")
    s = lax.axis_index("s")
    w = s * NC + c
    base = s * ROWS_PT
    pltpu.sync_copy(zeros_hbm.at[pl.ds(base, ROWS_PT)],
                    acc_sh.at[pl.ds(base, ROWS_PT)])
    off = NB_LO * w + jnp.maximum(0, w - W_HI)
    descs = []
    for k in range(NB_HI):
        for m in range(TPB):
            t = (off + k) * TPB + m
            descs.append(pltpu.async_copy(
                ei3.at[t, 0, :], src_v.at[k, pl.ds(m * 128, 128)], sem))
            descs.append(pltpu.async_copy(
                ei3.at[t, 1, :], dst_v.at[k, pl.ds(m * 128, 128)], sem))
    for d in descs:
        d.wait()
    pltpu.sync_copy(src_v, src_lin.at[pl.ds(off, NB_HI)])
    pltpu.sync_copy(dst_v, dst_lin.at[pl.ds(off, NB_HI)])
    pltpu.sync_copy(ones_hbm, ones_v)
    plsc.subcore_barrier()

    def body(j, carry):
        pltpu.sync_copy(ones_v, acc_sh.at[dst_v.at[j]], add=True)
        return carry

    nb = NB_LO + jnp.where(w >= W_HI, 1, 0)
    lax.fori_loop(0, nb, body, 0)
    plsc.subcore_barrier()
    pltpu.sync_copy(acc_sh.at[pl.ds(base, ROWS_PT)],
                    deg_out.at[c, pl.ds(base, ROWS_PT)])


def _make_agg_kernel(d):
    @functools.partial(
        pl.kernel,
        out_type=jax.ShapeDtypeStruct((NC, NPAD, d), jnp.float32),
        mesh=plsc.VectorSubcoreMesh(**_MESH),
        compiler_params=_SC_PARAMS,
        scratch_types=[
            pltpu.VMEM((NB_HI, EB), jnp.int32),
            pltpu.VMEM((NB_HI, EB), jnp.int32),
            pltpu.VMEM((EB, d), jnp.float32),
            pltpu.VMEM_SHARED((NPAD, d), jnp.float32),
            pltpu.SemaphoreType.DMA,
        ],
    )
    def _agg(hs_hbm, srcf, dstf, zeros_hbm, out,
             src_v, dst_v, rows_v, acc_sh, sem):
        c = lax.axis_index("c")
        s = lax.axis_index("s")
        w = s * NC + c
        base = s * ROWS_PT
        pltpu.sync_copy(zeros_hbm.at[pl.ds(base, ROWS_PT)],
                        acc_sh.at[pl.ds(base, ROWS_PT)])
        off = NB_LO * w + jnp.maximum(0, w - W_HI)
        pltpu.sync_copy(srcf.at[pl.ds(off, NB_HI)], src_v)
        pltpu.sync_copy(dstf.at[pl.ds(off, NB_HI)], dst_v)
        plsc.subcore_barrier()

        def body(j, carry):
            pltpu.async_copy(hs_hbm.at[src_v.at[j]], rows_v, sem).wait()
            pltpu.sync_copy(rows_v, acc_sh.at[dst_v.at[j]], add=True)
            return carry

        nb = NB_LO + jnp.where(w >= W_HI, 1, 0)
        lax.fori_loop(0, nb, body, 0)
        plsc.subcore_barrier()
        pltpu.sync_copy(acc_sh.at[pl.ds(base, ROWS_PT)],
                        out.at[c, pl.ds(base, ROWS_PT)])

    return _agg


D2 = 48
_agg16 = _make_agg_kernel(HID)
_agg48 = _make_agg_kernel(D2)


_RB = 2000


def _l1_body(x_ref, w_ref, deg_ref, hs_ref, dinv_ref):
    h1 = jnp.dot(x_ref[...], w_ref[...], preferred_element_type=jnp.float32)
    deg = deg_ref[0][:, :1] + deg_ref[1][:, :1] + 1.0
    dinv = lax.rsqrt(deg)
    dinv_ref[...] = dinv
    hs_ref[...] = h1 * dinv


def _mid_body(agg_ref, hs1_ref, dinv_ref, b1_ref, w2_ref, hs2_ref):
    dinv = dinv_ref[...]
    t = (agg_ref[0] + agg_ref[1] + hs1_ref[...]) * dinv + b1_ref[...]
    t = jnp.maximum(t, 0.0)
    hs2_ref[...] = jnp.dot(t, w2_ref[...],
                           preferred_element_type=jnp.float32) * dinv


def _out_body(agg_ref, hs2_ref, dinv_ref, b2_ref, o_ref):
    o_ref[...] = ((agg_ref[0][:, :OUT_DIM] + agg_ref[1][:, :OUT_DIM]
                   + hs2_ref[...][:, :OUT_DIM]) * dinv_ref[...]
                  + b2_ref[...])


def kernel(x, edge_index, W1, b1, W2, b2):
    ei = edge_index.astype(jnp.int32)
    ei3 = ei.reshape(2, TOTB * TPB, 128).transpose(1, 0, 2)

    W2p = jnp.pad(W2, ((0, 0), (0, D2 - OUT_DIM)))
    b1r = b1.reshape(1, HID)
    b2r = b2.reshape(1, OUT_DIM)

    ones_eb = jnp.ones((EB, DW), jnp.float32)
    zeros1 = jnp.zeros((NPAD, DW), jnp.float32)
    zeros16 = jnp.zeros((NPAD, HID), jnp.float32)
    zeros48 = jnp.zeros((NPAD, D2), jnp.float32)

    deg_parts, srcf, dstf = _deg_kernel(ei3, ones_eb, zeros1)

    hs1, dinv = pl.pallas_call(
        _l1_body,
        grid=(N_NODES // _RB,),
        in_specs=[pl.BlockSpec((_RB, IN_DIM), lambda i: (i, 0)),
                  pl.BlockSpec((IN_DIM, HID), lambda i: (0, 0)),
                  pl.BlockSpec((NC, _RB, DW), lambda i: (0, i, 0))],
        out_specs=[pl.BlockSpec((_RB, HID), lambda i: (i, 0)),
                   pl.BlockSpec((_RB, 1), lambda i: (i, 0))],
        out_shape=[jax.ShapeDtypeStruct((N_NODES, HID), jnp.float32),
                   jax.ShapeDtypeStruct((N_NODES, 1), jnp.float32)],
    )(x, W1, deg_parts)

    agg1 = _agg16(hs1, srcf, dstf, zeros16)

    hs2 = pl.pallas_call(
        _mid_body,
        grid=(N_NODES // _RB,),
        in_specs=[pl.BlockSpec((NC, _RB, HID), lambda i: (0, i, 0)),
                  pl.BlockSpec((_RB, HID), lambda i: (i, 0)),
                  pl.BlockSpec((_RB, 1), lambda i: (i, 0)),
                  pl.BlockSpec((1, HID), lambda i: (0, 0)),
                  pl.BlockSpec((HID, D2), lambda i: (0, 0))],
        out_specs=pl.BlockSpec((_RB, D2), lambda i: (i, 0)),
        out_shape=jax.ShapeDtypeStruct((N_NODES, D2), jnp.float32),
    )(agg1, hs1, dinv, b1r, W2p)

    agg2 = _agg48(hs2, srcf, dstf, zeros48)

    outp = pl.pallas_call(
        _out_body,
        grid=(N_NODES // _RB,),
        in_specs=[pl.BlockSpec((NC, _RB, D2), lambda i: (0, i, 0)),
                  pl.BlockSpec((_RB, D2), lambda i: (i, 0)),
                  pl.BlockSpec((_RB, 1), lambda i: (i, 0)),
                  pl.BlockSpec((1, OUT_DIM), lambda i: (0, 0))],
        out_specs=pl.BlockSpec((_RB, OUT_DIM), lambda i: (i, 0)),
        out_shape=jax.ShapeDtypeStruct((N_NODES, OUT_DIM), jnp.float32),
    )(agg2, hs2, dinv, b2r)

    return outp


# speedup vs baseline: 1.0945x; 1.0111x over previous
import functools

import jax
import jax.numpy as jnp
from jax import lax
from jax.experimental import pallas as pl
from jax.experimental.pallas import tpu as pltpu
from jax.experimental.pallas import tpu_sc as plsc

N_NODES = 10000
N_EDGES = 320000
IN_DIM = 128
HID = 16
OUT_DIM = 40

NC, NS = 2, 16
NW = NC * NS
NPAD = 10112
ROWS_PT = NPAD // NS
EB = 1280
TPB = EB // 128
TOTB = N_EDGES // EB
NB_LO = TOTB // NW
NB_HI = NB_LO + 1
W_HI = NW - (TOTB - NW * NB_LO)

_MESH = dict(core_axis_name="c", subcore_axis_name="s",
             num_cores=NC, num_subcores=NS)
_SC_PARAMS = pltpu.CompilerParams(use_tc_tiling_on_sc=False)


DW = 8


@functools.partial(
    pl.kernel,
    out_type=[jax.ShapeDtypeStruct((NC, NPAD, DW), jnp.float32),
              jax.ShapeDtypeStruct((TOTB, EB), jnp.int32),
              jax.ShapeDtypeStruct((TOTB, EB), jnp.int32)],
    mesh=plsc.VectorSubcoreMesh(**_MESH),
    compiler_params=_SC_PARAMS,
    scratch_types=[
        pltpu.VMEM((NB_HI, EB), jnp.int32),
        pltpu.VMEM((NB_HI, EB), jnp.int32),
        pltpu.VMEM((EB, DW), jnp.float32),
        pltpu.VMEM_SHARED((NPAD, DW), jnp.float32),
        pltpu.SemaphoreType.DMA,
    ],
)
def _deg_kernel(ei3, ones_hbm, zeros_hbm, deg_out, src_lin, dst_lin,
                src_v, dst_v, ones_v, acc_sh, sem):
    c = lax.axis_index("c")
    s = lax.axis_index("s")
    w = s * NC + c
    base = s * ROWS_PT
    pltpu.sync_copy(zeros_hbm.at[pl.ds(base, ROWS_PT)],
                    acc_sh.at[pl.ds(base, ROWS_PT)])
    off = NB_LO * w + jnp.maximum(0, w - W_HI)
    descs = []
    for k in range(NB_HI):
        for m in range(TPB):
            t = (off + k) * TPB + m
            descs.append(pltpu.async_copy(
                ei3.at[t, 0, :], src_v.at[k, pl.ds(m * 128, 128)], sem))
            descs.append(pltpu.async_copy(
                ei3.at[t, 1, :], dst_v.at[k, pl.ds(m * 128, 128)], sem))
    for d in descs:
        d.wait()
    pltpu.sync_copy(src_v, src_lin.at[pl.ds(off, NB_HI)])
    pltpu.sync_copy(dst_v, dst_lin.at[pl.ds(off, NB_HI)])
    pltpu.sync_copy(ones_hbm, ones_v)
    plsc.subcore_barrier()

    def body(j, carry):
        pltpu.sync_copy(ones_v, acc_sh.at[dst_v.at[j]], add=True)
        return carry

    nb = NB_LO + jnp.where(w >= W_HI, 1, 0)
    lax.fori_loop(0, nb, body, 0)
    plsc.subcore_barrier()
    pltpu.sync_copy(acc_sh.at[pl.ds(base, ROWS_PT)],
                    deg_out.at[c, pl.ds(base, ROWS_PT)])


def _make_agg_kernel(d):
    @functools.partial(
        pl.kernel,
        out_type=jax.ShapeDtypeStruct((NC, NPAD, d), jnp.float32),
        mesh=plsc.VectorSubcoreMesh(**_MESH),
        compiler_params=_SC_PARAMS,
        scratch_types=[
            pltpu.VMEM((NB_HI, EB), jnp.int32),
            pltpu.VMEM((NB_HI, EB), jnp.int32),
            pltpu.VMEM((EB, d), jnp.float32),
            pltpu.VMEM_SHARED((NPAD, d), jnp.float32),
            pltpu.SemaphoreType.DMA,
        ],
    )
    def _agg(hs_hbm, srcf, dstf, zeros_hbm, out,
             src_v, dst_v, rows_v, acc_sh, sem):
        c = lax.axis_index("c")
        s = lax.axis_index("s")
        w = s * NC + c
        base = s * ROWS_PT
        pltpu.sync_copy(zeros_hbm.at[pl.ds(base, ROWS_PT)],
                        acc_sh.at[pl.ds(base, ROWS_PT)])
        off = NB_LO * w + jnp.maximum(0, w - W_HI)
        pltpu.sync_copy(srcf.at[pl.ds(off, NB_HI)], src_v)
        pltpu.sync_copy(dstf.at[pl.ds(off, NB_HI)], dst_v)
        plsc.subcore_barrier()

        def body(j, carry):
            pltpu.async_copy(hs_hbm.at[src_v.at[j]], rows_v, sem).wait()
            pltpu.sync_copy(rows_v, acc_sh.at[dst_v.at[j]], add=True)
            return carry

        nb = NB_LO + jnp.where(w >= W_HI, 1, 0)
        lax.fori_loop(0, nb, body, 0)
        plsc.subcore_barrier()
        pltpu.sync_copy(acc_sh.at[pl.ds(base, ROWS_PT)],
                        out.at[c, pl.ds(base, ROWS_PT)])

    return _agg


D2 = 48
_agg16 = _make_agg_kernel(HID)
_agg48 = _make_agg_kernel(D2)


_RB = 5000


def _l1_body(x_ref, w_ref, deg_ref, hs_ref, dinv_ref):
    h1 = jnp.dot(x_ref[...], w_ref[...], preferred_element_type=jnp.float32)
    deg = deg_ref[0][:, :1] + deg_ref[1][:, :1] + 1.0
    dinv = lax.rsqrt(deg)
    dinv_ref[...] = dinv
    hs_ref[...] = h1 * dinv


def _mid_body(agg_ref, hs1_ref, dinv_ref, b1_ref, w2_ref, hs2_ref):
    dinv = dinv_ref[...]
    t = (agg_ref[0] + agg_ref[1] + hs1_ref[...]) * dinv + b1_ref[...]
    t = jnp.maximum(t, 0.0)
    hs2_ref[...] = jnp.dot(t, w2_ref[...],
                           preferred_element_type=jnp.float32) * dinv


def _out_body(agg_ref, hs2_ref, dinv_ref, b2_ref, o_ref):
    o_ref[...] = ((agg_ref[0][:, :OUT_DIM] + agg_ref[1][:, :OUT_DIM]
                   + hs2_ref[...][:, :OUT_DIM]) * dinv_ref[...]
                  + b2_ref[...])


def kernel(x, edge_index, W1, b1, W2, b2):
    ei = edge_index.astype(jnp.int32)
    ei3 = ei.reshape(2, TOTB * TPB, 128).transpose(1, 0, 2)

    W2p = jnp.pad(W2, ((0, 0), (0, D2 - OUT_DIM)))
    b1r = b1.reshape(1, HID)
    b2r = b2.reshape(1, OUT_DIM)

    ones_eb = jnp.ones((EB, DW), jnp.float32)
    zeros1 = jnp.zeros((NPAD, DW), jnp.float32)
    zeros16 = jnp.zeros((NPAD, HID), jnp.float32)
    zeros48 = jnp.zeros((NPAD, D2), jnp.float32)

    deg_parts, srcf, dstf = _deg_kernel(ei3, ones_eb, zeros1)

    hs1, dinv = pl.pallas_call(
        _l1_body,
        grid=(N_NODES // _RB,),
        in_specs=[pl.BlockSpec((_RB, IN_DIM), lambda i: (i, 0)),
                  pl.BlockSpec((IN_DIM, HID), lambda i: (0, 0)),
                  pl.BlockSpec((NC, _RB, DW), lambda i: (0, i, 0))],
        out_specs=[pl.BlockSpec((_RB, HID), lambda i: (i, 0)),
                   pl.BlockSpec((_RB, 1), lambda i: (i, 0))],
        out_shape=[jax.ShapeDtypeStruct((N_NODES, HID), jnp.float32),
                   jax.ShapeDtypeStruct((N_NODES, 1), jnp.float32)],
    )(x, W1, deg_parts)

    agg1 = _agg16(hs1, srcf, dstf, zeros16)

    hs2 = pl.pallas_call(
        _mid_body,
        grid=(N_NODES // _RB,),
        in_specs=[pl.BlockSpec((NC, _RB, HID), lambda i: (0, i, 0)),
                  pl.BlockSpec((_RB, HID), lambda i: (i, 0)),
                  pl.BlockSpec((_RB, 1), lambda i: (i, 0)),
                  pl.BlockSpec((1, HID), lambda i: (0, 0)),
                  pl.BlockSpec((HID, D2), lambda i: (0, 0))],
        out_specs=pl.BlockSpec((_RB, D2), lambda i: (i, 0)),
        out_shape=jax.ShapeDtypeStruct((N_NODES, D2), jnp.float32),
    )(agg1, hs1, dinv, b1r, W2p)

    agg2 = _agg48(hs2, srcf, dstf, zeros48)

    outp = pl.pallas_call(
        _out_body,
        grid=(N_NODES // _RB,),
        in_specs=[pl.BlockSpec((NC, _RB, D2), lambda i: (0, i, 0)),
                  pl.BlockSpec((_RB, D2), lambda i: (i, 0)),
                  pl.BlockSpec((_RB, 1), lambda i: (i, 0)),
                  pl.BlockSpec((1, OUT_DIM), lambda i: (0, 0))],
        out_specs=pl.BlockSpec((_RB, OUT_DIM), lambda i: (i, 0)),
        out_shape=jax.ShapeDtypeStruct((N_NODES, OUT_DIM), jnp.float32),
    )(agg2, hs2, dinv, b2r)

    return outp
